# baseline (speedup 1.0000x reference)
import jax
import jax.numpy as jnp
from jax import lax
from jax.experimental import pallas as pl
from jax.experimental.pallas import tpu as pltpu
from jax.experimental.pallas import tpu_sc as plsc

D = 64
B = 1024
S = 200
NT = 20
SG = S - NT
NC = 2
NS = 16
NW = NC * NS
BPW = B // NW
CH = 96
IW = 2 * CH
RV = NT + IW
NBUF = 6
GDEPTH = 3


def _sc_body(tok_hbm, wte_hbm, learned_hbm, out_hbm, idx_raw, idx_t, rows_v,
             gsem, osem):
    w = lax.axis_index("s") * NC + lax.axis_index("c")

    def drain_out():
        pltpu.make_async_copy(
            out_hbm.at[pl.ds(0, S)], rows_v.at[0, pl.ds(0, S)], osem).wait()

    def drain_gather():
        pltpu.make_async_copy(
            out_hbm.at[pl.ds(0, CH)], rows_v.at[0, pl.ds(NT, CH)], gsem).wait()

    pltpu.sync_copy(tok_hbm.at[pl.ds(NT, SG), pl.ds(w * BPW, BPW)], idx_raw)
    for k in range(NBUF):
        pltpu.sync_copy(learned_hbm, rows_v.at[k, pl.ds(0, NT)])

    lanes = lax.iota(jnp.int32, 16)

    def tbody(j, carry):
        colj = jnp.full((16,), j, jnp.int32)
        for k in range(IW // 16):
            rows = jnp.minimum(k * 16 + lanes, SG - 1)
            idx_t[j, pl.ds(k * 16, 16)] = plsc.load_gather(idx_raw, [rows, colj])
        return carry

    lax.fori_loop(0, BPW, tbody, 0)

    def body(j, carry):
        p = lax.rem(j, NBUF)

        @pl.when(j >= NBUF)
        def _():
            drain_out()

        @pl.when(j < BPW)
        def _():
            pltpu.async_copy(
                wte_hbm.at[idx_t.at[j, pl.ds(0, CH)]],
                rows_v.at[p, pl.ds(NT, CH)], gsem)
            pltpu.async_copy(
                wte_hbm.at[idx_t.at[j, pl.ds(CH, CH)]],
                rows_v.at[p, pl.ds(NT + CH, CH)], gsem)

        @pl.when(j >= GDEPTH)
        def _():
            t = j - GDEPTH
            q = lax.rem(t, NBUF)
            drain_gather()
            drain_gather()
            pltpu.async_copy(
                rows_v.at[q, pl.ds(0, S)],
                out_hbm.at[pl.ds((w * BPW + t) * S, S)], osem)

        return carry

    lax.fori_loop(0, BPW + GDEPTH, body, 0)
    for _ in range(NBUF - GDEPTH):
        drain_out()


@jax.jit
def _gather(tok_t, wte_weight, learned_embedding):
    mesh = plsc.VectorSubcoreMesh(core_axis_name="c", subcore_axis_name="s")
    return pl.kernel(
        _sc_body,
        out_type=jax.ShapeDtypeStruct((B * S, D), jnp.float32),
        mesh=mesh,
        scratch_types=[
            pltpu.VMEM((SG, BPW), jnp.int32),
            pltpu.VMEM((BPW, IW), jnp.int32),
            pltpu.VMEM((NBUF, RV, D), jnp.float32),
            pltpu.SemaphoreType.DMA,
            pltpu.SemaphoreType.DMA,
        ],
        compiler_params=pltpu.CompilerParams(
            use_tc_tiling_on_sc=False, needs_layout_passes=False),
    )(tok_t, wte_weight, learned_embedding)


def kernel(tokens, wte_weight, learned_embedding):
    tok_t = jnp.swapaxes(tokens, 0, 1)
    out = _gather(tok_t, wte_weight, learned_embedding)
    return out.reshape(B, S, D)
